# baseline (speedup 1.0000x reference)
import functools

import jax
import jax.numpy as jnp
from jax import lax
from jax.experimental import pallas as pl
from jax.experimental.pallas import tpu as pltpu
from jax.experimental.pallas import tpu_sc as plsc

_N = 10000
_E = 160000
_D = 256
_H = 128
_EP = 163840
_NCH = _EP // 128
_NS = 16
_EC = 128
_NCHS = _EP // _EC
_CPS = _NCHS // _NS
_TBL = 10112
_RPS = _TBL // _NS
_GARB = _N
_R = 2000


def _deg_sc(dst2d, zrows, ones128):
    mesh = plsc.VectorSubcoreMesh(core_axis_name="c", subcore_axis_name="s")

    @functools.partial(
        pl.kernel,
        out_type=jax.ShapeDtypeStruct((2 * _TBL, 128), jnp.float32),
        mesh=mesh,
        scratch_types=[
            pltpu.VMEM((_NCH // 32, 128), jnp.int32),
            pltpu.VMEM((128, 128), jnp.float32),
            pltpu.VMEM_SHARED((_TBL, 128), jnp.float32),
        ],
    )
    def k(dst_hbm, z_hbm, ones_hbm, deg_hbm, dst_v, ones_v, deg_sh):
        cid = lax.axis_index("c")
        sid = lax.axis_index("s")
        cps = _NCH // 32
        r0 = sid * _RPS
        pltpu.sync_copy(z_hbm, deg_sh.at[pl.ds(r0, _RPS)])
        pltpu.sync_copy(ones_hbm, ones_v)
        pltpu.sync_copy(dst_hbm.at[pl.ds(cid * (_NCH // 2) + sid * cps, cps)], dst_v)
        plsc.subcore_barrier()

        @pl.loop(0, cps)
        def _(j):
            pltpu.sync_copy(ones_v, deg_sh.at[dst_v.at[j]], add=True)

        plsc.subcore_barrier()
        pltpu.sync_copy(deg_sh.at[pl.ds(r0, _RPS)],
                        deg_hbm.at[pl.ds(cid * _TBL + r0, _RPS)])

    return k(dst2d, zrows, ones128)


def _scatter_sc(gcat, srcs2d, dst2d, zrows):
    mesh = plsc.VectorSubcoreMesh(core_axis_name="c", subcore_axis_name="s")

    nb = 2

    @functools.partial(
        pl.kernel,
        out_type=jax.ShapeDtypeStruct((2 * _TBL, _H), jnp.float32),
        mesh=mesh,
        scratch_types=(
            [pltpu.VMEM((_CPS, _EC), jnp.int32)]
            + [pltpu.VMEM((_EC,), jnp.int32)] * nb
            + [pltpu.VMEM((_EC, _H), jnp.float32)] * nb
            + [pltpu.SemaphoreType.DMA] * (3 * nb)
            + [pltpu.VMEM_SHARED((_TBL, _H), jnp.float32)]
        ),
    )
    def k(gcat_hbm, srcs_hbm, dst_hbm, z_hbm, u_hbm, src_v, *rest):
        didx = rest[:nb]
        rows = rest[nb:2 * nb]
        si = rest[2 * nb:3 * nb]
        sg = rest[3 * nb:4 * nb]
        ss = rest[4 * nb:5 * nb]
        u_sh = rest[5 * nb]
        cid = lax.axis_index("c")
        sid = lax.axis_index("s")
        r0 = sid * _RPS
        base = cid * _NCHS + sid * _CPS
        dbase = sid * _CPS
        pltpu.sync_copy(srcs_hbm.at[pl.ds(base, _CPS)], src_v)
        for b in range(nb):
            pltpu.async_copy(dst_hbm.at[dbase + b], didx[b], si[b])
        pltpu.sync_copy(z_hbm, u_sh.at[pl.ds(r0, _RPS)])
        plsc.subcore_barrier()

        for b in range(nb):
            pltpu.async_copy(gcat_hbm.at[src_v.at[b]], rows[b], sg[b])

        @pl.loop(0, _CPS, step=nb)
        def _(j):
            for b in range(nb):
                pltpu.make_async_copy(gcat_hbm.at[src_v.at[j + b]],
                                      rows[b], sg[b]).wait()
                pltpu.make_async_copy(dst_hbm.at[dbase + j + b],
                                      didx[b], si[b]).wait()
                pltpu.async_copy(rows[b], u_sh.at[didx[b]], ss[b], add=True)
            for b in range(nb):
                pltpu.make_async_copy(rows[b], u_sh.at[didx[b]], ss[b]).wait()

                @pl.when(j + nb + b < _CPS)
                def _():
                    pltpu.async_copy(gcat_hbm.at[src_v.at[j + nb + b]],
                                     rows[b], sg[b])
                    pltpu.async_copy(dst_hbm.at[dbase + j + nb + b],
                                     didx[b], si[b])

        plsc.subcore_barrier()
        pltpu.sync_copy(u_sh.at[pl.ds(r0, _RPS)],
                        u_hbm.at[pl.ds(cid * _TBL + r0, _RPS)])

    return k(gcat, srcs2d, dst2d, zrows)


def _dinv_of(d_ref):
    tot = d_ref[0, :, 0] + d_ref[1, :, 0] + 1.0
    return (1.0 / jnp.sqrt(tot))[:, None]


def _k0(x, W_in, b_in, W1, deg):
    def body(x_ref, wi_ref, bi_ref, w1_ref, d_ref, g_ref):
        dinv = _dinv_of(d_ref)
        t = jnp.maximum(
            jnp.dot(x_ref[...], wi_ref[...], preferred_element_type=jnp.float32)
            + bi_ref[...], 0.0)
        g = jnp.dot(t, w1_ref[...], preferred_element_type=jnp.float32) * dinv
        g_ref[0] = g[:, :_H]
        g_ref[1] = g[:, _H:]

    return pl.pallas_call(
        body,
        grid=(_N // _R,),
        in_specs=[
            pl.BlockSpec((_R, _D), lambda i: (i, 0)),
            pl.BlockSpec((_D, _D), lambda i: (0, 0)),
            pl.BlockSpec((1, _D), lambda i: (0, 0)),
            pl.BlockSpec((_D, _D), lambda i: (0, 0)),
            pl.BlockSpec((2, _R, 128), lambda i: (0, i, 0)),
        ],
        out_specs=[pl.BlockSpec((2, _R, _H), lambda i: (0, i, 0))],
        out_shape=[jax.ShapeDtypeStruct((2, _N, _H), jnp.float32)],
    )(x, W_in, b_in.reshape(1, _D), W1, deg)[0]


def _layer(u, g, deg, b, W_next):
    def body(u_ref, g_ref, d_ref, b_ref, wn_ref, h_ref, n_ref):
        dinv = _dinv_of(d_ref)
        left = (u_ref[0] + g_ref[0]) * dinv
        right = (u_ref[1] + g_ref[1]) * dinv
        h = jnp.maximum(jnp.concatenate([left, right], axis=1) + b_ref[...], 0.0)
        h_ref[...] = h
        gn = jnp.dot(h, wn_ref[...], preferred_element_type=jnp.float32) * dinv
        n_ref[0] = gn[:, :_H]
        n_ref[1] = gn[:, _H:]

    return pl.pallas_call(
        body,
        grid=(_N // _R,),
        in_specs=[
            pl.BlockSpec((2, _R, _H), lambda i: (0, i, 0)),
            pl.BlockSpec((2, _R, _H), lambda i: (0, i, 0)),
            pl.BlockSpec((2, _R, 128), lambda i: (0, i, 0)),
            pl.BlockSpec((1, _D), lambda i: (0, 0)),
            pl.BlockSpec((_D, _D), lambda i: (0, 0)),
        ],
        out_specs=[
            pl.BlockSpec((_R, _D), lambda i: (i, 0)),
            pl.BlockSpec((2, _R, _H), lambda i: (0, i, 0)),
        ],
        out_shape=[
            jax.ShapeDtypeStruct((_N, _D), jnp.float32),
            jax.ShapeDtypeStruct((2, _N, _H), jnp.float32),
        ],
    )(u, g, deg, b.reshape(1, _D), W_next)


def _layer_last(u, g, deg, b, W_cls, b_cls):
    def body(u_ref, g_ref, d_ref, b_ref, wc_ref, bc_ref, h_ref, y_ref):
        dinv = _dinv_of(d_ref)
        left = (u_ref[0] + g_ref[0]) * dinv
        right = (u_ref[1] + g_ref[1]) * dinv
        h = jnp.maximum(jnp.concatenate([left, right], axis=1) + b_ref[...], 0.0)
        h_ref[...] = h
        y_ref[...] = jnp.dot(h, wc_ref[...],
                             preferred_element_type=jnp.float32) + bc_ref[...]

    d_out = W_cls.shape[1]
    return pl.pallas_call(
        body,
        grid=(_N // _R,),
        in_specs=[
            pl.BlockSpec((2, _R, _H), lambda i: (0, i, 0)),
            pl.BlockSpec((2, _R, _H), lambda i: (0, i, 0)),
            pl.BlockSpec((2, _R, 128), lambda i: (0, i, 0)),
            pl.BlockSpec((1, _D), lambda i: (0, 0)),
            pl.BlockSpec((_D, d_out), lambda i: (0, 0)),
            pl.BlockSpec((1, d_out), lambda i: (0, 0)),
        ],
        out_specs=[
            pl.BlockSpec((_R, _D), lambda i: (i, 0)),
            pl.BlockSpec((_R, d_out), lambda i: (i, 0)),
        ],
        out_shape=[
            jax.ShapeDtypeStruct((_N, _D), jnp.float32),
            jax.ShapeDtypeStruct((_N, d_out), jnp.float32),
        ],
    )(u, g, deg, b.reshape(1, _D), W_cls, b_cls.reshape(1, d_out))


def kernel(x, edge_index, W_in, b_in, W1, b1, W2, b2, W3, b3, W4, b4, W_cls, b_cls):
    src = edge_index[0]
    dst = edge_index[1]
    pad = _EP - _E
    src_p = jnp.concatenate([src, jnp.zeros((pad,), jnp.int32)])
    dst_p = jnp.concatenate([dst, jnp.full((pad,), _GARB, jnp.int32)])
    src2d = src_p.reshape(_NCHS, _EC)
    srcs2d = jnp.concatenate([src2d, src2d + _N], axis=0)
    dst2d = dst_p.reshape(_NCHS, _EC)
    dstdeg = dst_p.reshape(_NCH, 128)
    zrows = jnp.zeros((_RPS, _H), jnp.float32)
    ones128 = jnp.ones((128, 128), jnp.float32)

    deg = _deg_sc(dstdeg, zrows, ones128).reshape(2, _TBL, 128)
    g = _k0(x, W_in, b_in, W1, deg)

    def sc(gx):
        return _scatter_sc(gx.reshape(2 * _N, _H), srcs2d, dst2d,
                           zrows).reshape(2, _TBL, _H)

    h1, g = _layer(sc(g), g, deg, b1, W2)
    h2, g = _layer(sc(g), g, deg, b2, W3)
    h3, g = _layer(sc(g), g, deg, b3, W4)
    h4, y = _layer_last(sc(g), g, deg, b4, W_cls, b_cls)
    return h1, h2, h3, h4, y


# speedup vs baseline: 1.0059x; 1.0059x over previous
import functools

import jax
import jax.numpy as jnp
from jax import lax
from jax.experimental import pallas as pl
from jax.experimental.pallas import tpu as pltpu
from jax.experimental.pallas import tpu_sc as plsc

_N = 10000
_E = 160000
_D = 256
_H = 128
_EP = 163840
_NCH = _EP // 128
_NS = 16
_EC = 128
_NCHS = _EP // _EC
_CPS = _NCHS // _NS
_TBL = 10112
_RPS = _TBL // _NS
_GARB = _N
_R = 1000


def _deg_sc(dst2d, zrows, ones128):
    mesh = plsc.VectorSubcoreMesh(core_axis_name="c", subcore_axis_name="s")

    @functools.partial(
        pl.kernel,
        out_type=jax.ShapeDtypeStruct((2 * _TBL, 128), jnp.float32),
        mesh=mesh,
        scratch_types=[
            pltpu.VMEM((_NCH // 32, 128), jnp.int32),
            pltpu.VMEM((128, 128), jnp.float32),
            pltpu.VMEM_SHARED((_TBL, 128), jnp.float32),
        ],
    )
    def k(dst_hbm, z_hbm, ones_hbm, deg_hbm, dst_v, ones_v, deg_sh):
        cid = lax.axis_index("c")
        sid = lax.axis_index("s")
        cps = _NCH // 32
        r0 = sid * _RPS
        pltpu.sync_copy(z_hbm, deg_sh.at[pl.ds(r0, _RPS)])
        pltpu.sync_copy(ones_hbm, ones_v)
        pltpu.sync_copy(dst_hbm.at[pl.ds(cid * (_NCH // 2) + sid * cps, cps)], dst_v)
        plsc.subcore_barrier()

        @pl.loop(0, cps)
        def _(j):
            pltpu.sync_copy(ones_v, deg_sh.at[dst_v.at[j]], add=True)

        plsc.subcore_barrier()
        pltpu.sync_copy(deg_sh.at[pl.ds(r0, _RPS)],
                        deg_hbm.at[pl.ds(cid * _TBL + r0, _RPS)])

    return k(dst2d, zrows, ones128)


def _scatter_sc(gcat, srcs2d, dst2d, zrows):
    mesh = plsc.VectorSubcoreMesh(core_axis_name="c", subcore_axis_name="s")

    nb = 2

    @functools.partial(
        pl.kernel,
        out_type=jax.ShapeDtypeStruct((2 * _TBL, _H), jnp.float32),
        mesh=mesh,
        scratch_types=(
            [pltpu.VMEM((_CPS, _EC), jnp.int32)]
            + [pltpu.VMEM((_EC,), jnp.int32)] * nb
            + [pltpu.VMEM((_EC, _H), jnp.float32)] * nb
            + [pltpu.SemaphoreType.DMA] * (3 * nb)
            + [pltpu.VMEM_SHARED((_TBL, _H), jnp.float32)]
        ),
    )
    def k(gcat_hbm, srcs_hbm, dst_hbm, z_hbm, u_hbm, src_v, *rest):
        didx = rest[:nb]
        rows = rest[nb:2 * nb]
        si = rest[2 * nb:3 * nb]
        sg = rest[3 * nb:4 * nb]
        ss = rest[4 * nb:5 * nb]
        u_sh = rest[5 * nb]
        cid = lax.axis_index("c")
        sid = lax.axis_index("s")
        r0 = sid * _RPS
        base = cid * _NCHS + sid * _CPS
        dbase = sid * _CPS
        pltpu.sync_copy(srcs_hbm.at[pl.ds(base, _CPS)], src_v)
        for b in range(nb):
            pltpu.async_copy(dst_hbm.at[dbase + b], didx[b], si[b])
        pltpu.sync_copy(z_hbm, u_sh.at[pl.ds(r0, _RPS)])
        plsc.subcore_barrier()

        for b in range(nb):
            pltpu.async_copy(gcat_hbm.at[src_v.at[b]], rows[b], sg[b])

        @pl.loop(0, _CPS, step=nb)
        def _(j):
            for b in range(nb):
                pltpu.make_async_copy(gcat_hbm.at[src_v.at[j + b]],
                                      rows[b], sg[b]).wait()
                pltpu.make_async_copy(dst_hbm.at[dbase + j + b],
                                      didx[b], si[b]).wait()
                pltpu.async_copy(rows[b], u_sh.at[didx[b]], ss[b], add=True)
            for b in range(nb):
                pltpu.make_async_copy(rows[b], u_sh.at[didx[b]], ss[b]).wait()

                @pl.when(j + nb + b < _CPS)
                def _():
                    pltpu.async_copy(gcat_hbm.at[src_v.at[j + nb + b]],
                                     rows[b], sg[b])
                    pltpu.async_copy(dst_hbm.at[dbase + j + nb + b],
                                     didx[b], si[b])

        plsc.subcore_barrier()
        pltpu.sync_copy(u_sh.at[pl.ds(r0, _RPS)],
                        u_hbm.at[pl.ds(cid * _TBL + r0, _RPS)])

    return k(gcat, srcs2d, dst2d, zrows)


def _dinv_of(d_ref):
    tot = d_ref[0, :, 0] + d_ref[1, :, 0] + 1.0
    return (1.0 / jnp.sqrt(tot))[:, None]


def _k0(x, W_in, b_in, W1, deg):
    def body(x_ref, wi_ref, bi_ref, w1_ref, d_ref, g_ref):
        dinv = _dinv_of(d_ref)
        t = jnp.maximum(
            jnp.dot(x_ref[...], wi_ref[...], preferred_element_type=jnp.float32)
            + bi_ref[...], 0.0)
        g = jnp.dot(t, w1_ref[...], preferred_element_type=jnp.float32) * dinv
        g_ref[0] = g[:, :_H]
        g_ref[1] = g[:, _H:]

    return pl.pallas_call(
        body,
        grid=(_N // _R,),
        in_specs=[
            pl.BlockSpec((_R, _D), lambda i: (i, 0)),
            pl.BlockSpec((_D, _D), lambda i: (0, 0)),
            pl.BlockSpec((1, _D), lambda i: (0, 0)),
            pl.BlockSpec((_D, _D), lambda i: (0, 0)),
            pl.BlockSpec((2, _R, 128), lambda i: (0, i, 0)),
        ],
        out_specs=[pl.BlockSpec((2, _R, _H), lambda i: (0, i, 0))],
        out_shape=[jax.ShapeDtypeStruct((2, _N, _H), jnp.float32)],
    )(x, W_in, b_in.reshape(1, _D), W1, deg)[0]


def _layer(u, g, deg, b, W_next):
    def body(u_ref, g_ref, d_ref, b_ref, wn_ref, h_ref, n_ref):
        dinv = _dinv_of(d_ref)
        left = (u_ref[0] + g_ref[0]) * dinv
        right = (u_ref[1] + g_ref[1]) * dinv
        h = jnp.maximum(jnp.concatenate([left, right], axis=1) + b_ref[...], 0.0)
        h_ref[...] = h
        gn = jnp.dot(h, wn_ref[...], preferred_element_type=jnp.float32) * dinv
        n_ref[0] = gn[:, :_H]
        n_ref[1] = gn[:, _H:]

    return pl.pallas_call(
        body,
        grid=(_N // _R,),
        in_specs=[
            pl.BlockSpec((2, _R, _H), lambda i: (0, i, 0)),
            pl.BlockSpec((2, _R, _H), lambda i: (0, i, 0)),
            pl.BlockSpec((2, _R, 128), lambda i: (0, i, 0)),
            pl.BlockSpec((1, _D), lambda i: (0, 0)),
            pl.BlockSpec((_D, _D), lambda i: (0, 0)),
        ],
        out_specs=[
            pl.BlockSpec((_R, _D), lambda i: (i, 0)),
            pl.BlockSpec((2, _R, _H), lambda i: (0, i, 0)),
        ],
        out_shape=[
            jax.ShapeDtypeStruct((_N, _D), jnp.float32),
            jax.ShapeDtypeStruct((2, _N, _H), jnp.float32),
        ],
    )(u, g, deg, b.reshape(1, _D), W_next)


def _layer_last(u, g, deg, b, W_cls, b_cls):
    def body(u_ref, g_ref, d_ref, b_ref, wc_ref, bc_ref, h_ref, y_ref):
        dinv = _dinv_of(d_ref)
        left = (u_ref[0] + g_ref[0]) * dinv
        right = (u_ref[1] + g_ref[1]) * dinv
        h = jnp.maximum(jnp.concatenate([left, right], axis=1) + b_ref[...], 0.0)
        h_ref[...] = h
        y_ref[...] = jnp.dot(h, wc_ref[...],
                             preferred_element_type=jnp.float32) + bc_ref[...]

    d_out = W_cls.shape[1]
    return pl.pallas_call(
        body,
        grid=(_N // _R,),
        in_specs=[
            pl.BlockSpec((2, _R, _H), lambda i: (0, i, 0)),
            pl.BlockSpec((2, _R, _H), lambda i: (0, i, 0)),
            pl.BlockSpec((2, _R, 128), lambda i: (0, i, 0)),
            pl.BlockSpec((1, _D), lambda i: (0, 0)),
            pl.BlockSpec((_D, d_out), lambda i: (0, 0)),
            pl.BlockSpec((1, d_out), lambda i: (0, 0)),
        ],
        out_specs=[
            pl.BlockSpec((_R, _D), lambda i: (i, 0)),
            pl.BlockSpec((_R, d_out), lambda i: (i, 0)),
        ],
        out_shape=[
            jax.ShapeDtypeStruct((_N, _D), jnp.float32),
            jax.ShapeDtypeStruct((_N, d_out), jnp.float32),
        ],
    )(u, g, deg, b.reshape(1, _D), W_cls, b_cls.reshape(1, d_out))


def kernel(x, edge_index, W_in, b_in, W1, b1, W2, b2, W3, b3, W4, b4, W_cls, b_cls):
    src = edge_index[0]
    dst = edge_index[1]
    pad = _EP - _E
    src_p = jnp.concatenate([src, jnp.zeros((pad,), jnp.int32)])
    dst_p = jnp.concatenate([dst, jnp.full((pad,), _GARB, jnp.int32)])
    src2d = src_p.reshape(_NCHS, _EC)
    srcs2d = jnp.concatenate([src2d, src2d + _N], axis=0)
    dst2d = dst_p.reshape(_NCHS, _EC)
    dstdeg = dst_p.reshape(_NCH, 128)
    zrows = jnp.zeros((_RPS, _H), jnp.float32)
    ones128 = jnp.ones((128, 128), jnp.float32)

    deg = _deg_sc(dstdeg, zrows, ones128).reshape(2, _TBL, 128)
    g = _k0(x, W_in, b_in, W1, deg)

    def sc(gx):
        return _scatter_sc(gx.reshape(2 * _N, _H), srcs2d, dst2d,
                           zrows).reshape(2, _TBL, _H)

    h1, g = _layer(sc(g), g, deg, b1, W2)
    h2, g = _layer(sc(g), g, deg, b2, W3)
    h3, g = _layer(sc(g), g, deg, b3, W4)
    h4, y = _layer_last(sc(g), g, deg, b4, W_cls, b_cls)
    return h1, h2, h3, h4, y
